# baseline (speedup 1.0000x reference)
import jax
import jax.numpy as jnp
from jax.experimental import pallas as pl
from jax.experimental.pallas import tpu as pltpu

NUM_CLASSES = 1000
B0 = 4096
B1 = 20
BLOCK = 128
NUM_BLOCKS = B0 // BLOCK


def _one_hot_kernel(idx_ref, out_ref):
    idx = idx_ref[...]
    classes = jax.lax.broadcasted_iota(
        jnp.int32, (BLOCK, B1, NUM_CLASSES), 2)
    out_ref[...] = (idx[:, :, None] == classes).astype(jnp.float32)


def kernel(inputs):
    idx = inputs.astype(jnp.int32)
    out = pl.pallas_call(
        _one_hot_kernel,
        grid=(NUM_BLOCKS,),
        in_specs=[pl.BlockSpec((BLOCK, B1), lambda i: (i, 0))],
        out_specs=pl.BlockSpec((BLOCK, B1, NUM_CLASSES), lambda i: (i, 0, 0)),
        out_shape=jax.ShapeDtypeStruct((B0, B1, NUM_CLASSES), jnp.float32),
        compiler_params=pltpu.CompilerParams(
            dimension_semantics=("parallel",)),
    )(idx)
    return out
